# baseline (speedup 1.0000x reference)
import functools
import math

import jax
import jax.numpy as jnp
from jax import lax
from jax.experimental import pallas as pl
from jax.experimental.pallas import tpu as pltpu
from jax.experimental.pallas import tpu_sc as plsc

D_MODEL = 512
EMB_SCALE = math.sqrt(float(D_MODEL))
N_FORMULA = 50


def _tc_matmul_into(spec, w, b, f, bb=8):
    bsz, t, d = spec.shape

    def body(s_ref, w_ref, b_ref, o_ref):
        s2 = s_ref[...].reshape(bb * t, d)
        m = jnp.dot(s2, w_ref[...], preferred_element_type=jnp.float32)
        o_ref[:, f:, :] = (m + b_ref[...]).reshape(bb, t, d)

    return pl.pallas_call(
        body,
        grid=(bsz // bb,),
        in_specs=[
            pl.BlockSpec((bb, t, d), lambda i: (i, 0, 0)),
            pl.BlockSpec((d, d), lambda i: (0, 0)),
            pl.BlockSpec((1, d), lambda i: (0, 0)),
        ],
        out_specs=pl.BlockSpec((bb, f + t, d), lambda i: (i, 0, 0)),
        out_shape=jax.ShapeDtypeStruct((bsz, f + t, d), jnp.float32),
    )(spec, w, b)


def _sc_scatter_embed(table, formula_pad, f, out_ref):
    bsz, f_pad = formula_pad.shape
    d = table.shape[1]
    f_up = 56
    info = plsc.get_sparse_core_info()
    num_workers = info.num_cores * info.num_subcores
    b_per_w = bsz // num_workers
    mesh = plsc.VectorSubcoreMesh(core_axis_name="c", subcore_axis_name="s")

    @functools.partial(
        pl.kernel,
        mesh=mesh,
        out_type=(),
        scratch_types=[
            pltpu.VMEM((b_per_w, f_pad), jnp.int32),
            pltpu.VMEM((f_up, d), jnp.float32),
            pltpu.VMEM((8, d), jnp.float32),
            pltpu.VMEM((f_up, d), jnp.float32),
            pltpu.SemaphoreType.DMA,
        ],
    )
    def k(table_hbm, formula_hbm, out_hbm, idx_v, gath_v, tail_v, st_v, sem):
        wid = lax.axis_index("s") * info.num_cores + lax.axis_index("c")
        b0 = wid * b_per_w
        pltpu.sync_copy(formula_hbm.at[pl.ds(b0, b_per_w)], idx_v)

        def one_batch(kk, _):
            b = b0 + kk
            pltpu.async_copy(
                table_hbm.at[idx_v.at[kk, pl.ds(0, f_up)]], gath_v, sem
            ).wait()
            pltpu.sync_copy(out_hbm.at[b, pl.ds(f_up - 8, 8)], tail_v)

            def scale_row(r, _):
                for c in range(d // 16):
                    sl = pl.ds(c * 16, 16)
                    st_v[r, sl] = gath_v[r, sl] * EMB_SCALE
                return 0

            lax.fori_loop(0, f, scale_row, 0)
            for j in range(f_up - f):
                for c in range(d // 16):
                    sl = pl.ds(c * 16, 16)
                    st_v[f + j, sl] = tail_v[8 - (f_up - f) + j, sl]
            pltpu.sync_copy(st_v, out_hbm.at[b, pl.ds(0, f_up)])
            return 0

        lax.fori_loop(0, b_per_w, one_batch, 0)

    k(table, formula_pad, out_ref)


def kernel(formula, spec, formula_table, W_spec, b_spec):
    bsz, f = formula.shape
    d = D_MODEL

    out0 = _tc_matmul_into(spec, W_spec, b_spec.reshape(1, d), f)
    ref = jax.new_ref(out0)
    formula_pad = jnp.pad(formula.astype(jnp.int32), ((0, 0), (0, 64 - f)))
    _sc_scatter_embed(formula_table, formula_pad, f, ref)
    return ref[...]


# speedup vs baseline: 1.0182x; 1.0182x over previous
import functools
import math

import jax
import jax.numpy as jnp
from jax import lax
from jax.experimental import pallas as pl
from jax.experimental.pallas import tpu as pltpu
from jax.experimental.pallas import tpu_sc as plsc

D_MODEL = 512
EMB_SCALE = math.sqrt(float(D_MODEL))


def _tc_matmul_into(spec, w, b, f, bb=16):
    bsz, t, d = spec.shape

    def body(s_ref, w_ref, b_ref, o_ref):
        s2 = s_ref[...].reshape(bb * t, d)
        m = jnp.dot(s2, w_ref[...], preferred_element_type=jnp.float32)
        o_ref[:, f:, :] = (m + b_ref[...]).reshape(bb, t, d)

    return pl.pallas_call(
        body,
        grid=(bsz // bb,),
        in_specs=[
            pl.BlockSpec((bb, t, d), lambda i: (i, 0, 0)),
            pl.BlockSpec((d, d), lambda i: (0, 0)),
            pl.BlockSpec((1, d), lambda i: (0, 0)),
        ],
        out_specs=pl.BlockSpec((bb, f + t, d), lambda i: (i, 0, 0)),
        out_shape=jax.ShapeDtypeStruct((bsz, f + t, d), jnp.float32),
    )(spec, w, b)


def _sc_scatter_embed(table, idx_flat, f, f_up, out_ref):
    d = table.shape[1]
    bsz = idx_flat.shape[0] // f_up
    info = plsc.get_sparse_core_info()
    num_workers = info.num_cores * info.num_subcores
    b_per_w = bsz // num_workers
    cb = 4
    n_chunks = b_per_w // cb
    rows_c = cb * f_up
    mesh = plsc.VectorSubcoreMesh(core_axis_name="c", subcore_axis_name="s")

    @functools.partial(
        pl.kernel,
        mesh=mesh,
        out_type=(),
        scratch_types=[
            pltpu.VMEM((rows_c,), jnp.int32),
            pltpu.VMEM((cb, f_up, d), jnp.float32),
            pltpu.VMEM((8, d), jnp.float32),
            pltpu.SemaphoreType.DMA,
        ],
    )
    def k(table_hbm, idx_hbm, out_hbm, idx_v, gath_v, tail_v, sem):
        wid = lax.axis_index("s") * info.num_cores + lax.axis_index("c")
        base_i = wid * b_per_w * f_up
        b0 = wid * b_per_w

        def one_chunk(c, _):
            pltpu.sync_copy(idx_hbm.at[pl.ds(base_i + c * rows_c, rows_c)], idx_v)
            gflat = gath_v.reshape(rows_c, d)
            pltpu.async_copy(table_hbm.at[idx_v], gflat, sem).wait()

            def scale_row(r, _):
                for cc in range(d // 16):
                    sl = pl.ds(cc * 16, 16)
                    gflat[r, sl] = gflat[r, sl] * EMB_SCALE
                return 0

            lax.fori_loop(0, rows_c, scale_row, 0)

            for kk in range(cb):
                pltpu.sync_copy(
                    out_hbm.at[b0 + c * cb + kk, pl.ds(f_up - 8, 8)], tail_v
                )
                for j in range(f_up - f):
                    for cc in range(d // 16):
                        sl = pl.ds(cc * 16, 16)
                        gath_v[kk, f + j, sl] = tail_v[8 - (f_up - f) + j, sl]

            pltpu.sync_copy(
                gath_v, out_hbm.at[pl.ds(b0 + c * cb, cb), pl.ds(0, f_up)]
            )
            return 0

        lax.fori_loop(0, n_chunks, one_chunk, 0)

    k(table, idx_flat, out_ref)


def kernel(formula, spec, formula_table, W_spec, b_spec):
    bsz, f = formula.shape
    d = D_MODEL
    f_up = 56

    out0 = _tc_matmul_into(spec, W_spec, b_spec.reshape(1, d), f)
    ref = jax.new_ref(out0)
    idx_flat = jnp.pad(
        formula.astype(jnp.int32), ((0, 0), (0, f_up - f))
    ).reshape(-1)
    _sc_scatter_embed(formula_table, idx_flat, f, f_up, ref)
    return ref[...]


# speedup vs baseline: 2.5338x; 2.4885x over previous
import functools
import math

import jax
import jax.numpy as jnp
from jax import lax
from jax.experimental import pallas as pl
from jax.experimental.pallas import tpu as pltpu
from jax.experimental.pallas import tpu_sc as plsc

D_MODEL = 512
EMB_SCALE = math.sqrt(float(D_MODEL))


def _sc_gather_tokmajor(table, idx_groups, f, bsz, t):
    d = table.shape[1]
    info = plsc.get_sparse_core_info()
    num_workers = info.num_cores * info.num_subcores
    n_groups = bsz // 8
    g_per_w = n_groups // num_workers
    pc = f // 2
    rows_c = pc * 8
    mesh = plsc.VectorSubcoreMesh(core_axis_name="c", subcore_axis_name="s")

    @functools.partial(
        pl.kernel,
        mesh=mesh,
        out_type=jax.ShapeDtypeStruct((f + t, bsz, d), jnp.float32),
        scratch_types=[
            pltpu.VMEM((rows_c,), jnp.int32),
            pltpu.VMEM((rows_c, d), jnp.float32),
            pltpu.SemaphoreType.DMA,
        ],
    )
    def k(table_hbm, idx_hbm, out_hbm, idx_v, gath_v, sem):
        wid = lax.axis_index("s") * info.num_cores + lax.axis_index("c")

        def one_chunk(c, _):
            g = wid * g_per_w + c // 2
            half = c % 2
            pltpu.sync_copy(
                idx_hbm.at[pl.ds(g * (f * 8) + half * rows_c, rows_c)], idx_v
            )
            pltpu.async_copy(table_hbm.at[idx_v], gath_v, sem).wait()

            def scale_row(r, _):
                for cc in range(d // 16):
                    sl = pl.ds(cc * 16, 16)
                    gath_v[r, sl] = gath_v[r, sl] * EMB_SCALE
                return 0

            lax.fori_loop(0, rows_c, scale_row, 0)
            pltpu.sync_copy(
                gath_v.reshape(pc, 8, d),
                out_hbm.at[pl.ds(half * pc, pc), pl.ds(g * 8, 8)],
            )
            return 0

        lax.fori_loop(0, 2 * g_per_w, one_chunk, 0)

    return k(table, idx_groups)


def _tc_matmul_tokmajor(spec, w, b, outT0, f):
    bsz, t, d = spec.shape
    bb = 256
    nj = t // 8
    ni = bsz // bb
    n_steps = ni * nj
    ring = 3

    def body(o_alias, s_hbm, w_ref, b_ref, o_hbm, s_bufs, o_bufs, fsems, wsems):
        step = pl.program_id(0)
        i = step // nj
        j = step % nj

        def fetch(s, q):
            pltpu.make_async_copy(
                s_hbm.at[pl.ds((s // nj) * bb, bb), pl.ds((s % nj) * 8, 8)],
                s_bufs[q],
                fsems[q],
            ).start()

        @pl.when(step == 0)
        def _():
            for q in range(ring - 1):
                fetch(step + q, q)

        fslot = step % ring
        oslot = step % 2

        def do_step(qf):
            pltpu.make_async_copy(
                s_hbm.at[pl.ds(0, bb), pl.ds(0, 8)], s_bufs[qf], fsems[qf]
            ).wait()

            @pl.when(step + ring - 1 < n_steps)
            def _():
                fetch(step + ring - 1, (qf + ring - 1) % ring)

            m = jnp.dot(
                s_bufs[qf][...].reshape(bb * 8, d),
                w_ref[...],
                preferred_element_type=jnp.float32,
            )
            m3 = (m + b_ref[...]).reshape(bb, 8, d)

            def write_out(qo):
                @pl.when(step >= 2)
                def _():
                    pltpu.make_async_copy(
                        o_bufs[qo], o_hbm.at[pl.ds(0, 8), pl.ds(0, bb)], wsems[qo]
                    ).wait()

                for tt in range(8):
                    o_bufs[qo][tt, :, :] = m3[:, tt, :]
                pltpu.make_async_copy(
                    o_bufs[qo],
                    o_hbm.at[pl.ds(f + j * 8, 8), pl.ds(i * bb, bb)],
                    wsems[qo],
                ).start()

            for qo in range(2):
                @pl.when(oslot == qo)
                def _(qo=qo):
                    write_out(qo)

        for qf in range(ring):
            @pl.when(fslot == qf)
            def _(qf=qf):
                do_step(qf)

        @pl.when(step == n_steps - 1)
        def _():
            for qo in range(2):
                pltpu.make_async_copy(
                    o_bufs[qo], o_hbm.at[pl.ds(0, 8), pl.ds(0, bb)], wsems[qo]
                ).wait()

    return pl.pallas_call(
        body,
        grid=(n_steps,),
        in_specs=[
            pl.BlockSpec(memory_space=pl.ANY),
            pl.BlockSpec(memory_space=pl.ANY),
            pl.BlockSpec((d, d), lambda s: (0, 0)),
            pl.BlockSpec((1, d), lambda s: (0, 0)),
        ],
        out_specs=pl.BlockSpec(memory_space=pl.ANY),
        out_shape=jax.ShapeDtypeStruct((f + t, bsz, d), jnp.float32),
        scratch_shapes=[
            [pltpu.VMEM((bb, 8, d), jnp.float32) for _ in range(3)],
            [pltpu.VMEM((8, bb, d), jnp.float32) for _ in range(2)],
            [pltpu.SemaphoreType.DMA for _ in range(3)],
            [pltpu.SemaphoreType.DMA for _ in range(2)],
        ],
        input_output_aliases={0: 0},
    )(outT0, spec, w, b)


def kernel(formula, spec, formula_table, W_spec, b_spec):
    bsz, f = formula.shape
    d = D_MODEL
    t = spec.shape[1]

    idx_groups = (
        formula.astype(jnp.int32)
        .reshape(bsz // 8, 8, f)
        .transpose(0, 2, 1)
        .reshape(-1)
    )
    outT0 = _sc_gather_tokmajor(formula_table, idx_groups, f, bsz, t)
    outT = _tc_matmul_tokmajor(spec, W_spec, b_spec.reshape(1, d), outT0, f)
    return jnp.transpose(outT, (1, 0, 2))
